# baseline (speedup 1.0000x reference)
import functools

import jax
import jax.numpy as jnp
from jax import lax
from jax.experimental import pallas as pl
from jax.experimental.pallas import tpu as pltpu
from jax.experimental.pallas import tpu_sc as plsc

EMB = 128
MAX_LEN = 512
BATCH = 16384
HIST = 200

NUM_CORES = 2
NUM_SUBCORES = 16
NUM_WORKERS = NUM_CORES * NUM_SUBCORES
LANES = 16

ROWS_PER_WORKER = BATCH // NUM_WORKERS
SLICE_OFFS = tuple(range(0, HIST - LANES + 1, LANES)) + (HIST - LANES,)


def _vgather(v, idx):
    dnums = lax.GatherDimensionNumbers(
        offset_dims=(), collapsed_slice_dims=(0,), start_index_map=(0,))
    return lax.gather(v, idx[:, None], dnums, (1,),
                      mode=lax.GatherScatterMode.PROMISE_IN_BOUNDS)


def _make_sc_kernel():
    mesh = plsc.VectorSubcoreMesh(core_axis_name="c", subcore_axis_name="s")

    @functools.partial(
        pl.kernel,
        mesh=mesh,
        compiler_params=pltpu.CompilerParams(needs_layout_passes=False),
        out_type=jax.ShapeDtypeStruct((BATCH * HIST, EMB), jnp.float32),
        scratch_types=[
            pltpu.VMEM((MAX_LEN, EMB), jnp.float32),
            pltpu.VMEM((HIST,), jnp.int32),
            pltpu.VMEM((HIST,), jnp.int32),
            pltpu.VMEM((HIST, EMB), jnp.float32),
            pltpu.VMEM((HIST, EMB), jnp.float32),
            pltpu.VMEM((HIST,), jnp.int32),
            pltpu.SemaphoreType.DMA,
            pltpu.SemaphoreType.DMA,
            pltpu.SemaphoreType.DMA,
            pltpu.SemaphoreType.DMA,
        ],
    )
    def sc_embed(vo_hbm, pe_hbm, out_hbm, pe_l, idx0, idx1, out0, out1,
                 norm, si0, si1, so0, so1):
        wid = lax.axis_index("s") * NUM_CORES + lax.axis_index("c")
        wbase = wid * ROWS_PER_WORKER
        zeros16 = jnp.zeros((LANES,), jnp.int32)
        iota16 = lax.iota(jnp.int32, LANES)
        cols = [iota16 + c * LANES for c in range(EMB // LANES)]
        idx_b = (idx0, idx1)
        out_b = (out0, out1)
        sem_i = (si0, si1)
        sem_o = (so0, so1)

        pltpu.sync_copy(pe_hbm, pe_l)

        def issue_idx(chunk, b):
            pltpu.async_copy(
                vo_hbm.at[pl.ds((wbase + chunk) * HIST, HIST)], idx_b[b],
                sem_i[b])

        def wait_idx(b):
            pltpu.make_async_copy(
                vo_hbm.at[pl.ds(0, HIST)], idx_b[b], sem_i[b]).wait()

        def issue_store(chunk, b):
            pltpu.async_copy(
                out_b[b], out_hbm.at[pl.ds((wbase + chunk) * HIST, HIST)],
                sem_o[b])

        def wait_store(b):
            pltpu.make_async_copy(
                out_b[b], out_hbm.at[pl.ds(0, HIST)], sem_o[b]).wait()

        def process(b):
            raw = idx_b[b]
            out_ref = out_b[b]
            f0 = _vgather(raw[pl.ds(0, LANES)], zeros16)
            for lo in SLICE_OFFS:
                v = raw[pl.ds(lo, LANES)]
                norm[pl.ds(lo, LANES)] = jnp.clip(v - f0, 0, MAX_LEN - 1)

            @plsc.parallel_loop(0, HIST, unroll=8)
            def entry_body(e):
                e_vec = jnp.full((LANES,), e, jnp.int32)
                r = plsc.load_gather(norm, [e_vec])
                for c in range(EMB // LANES):
                    val = plsc.load_gather(pe_l, [r, cols[c]])
                    plsc.store_scatter(out_ref, [e_vec, cols[c]], val)

        issue_idx(0, 0)
        issue_idx(1, 1)
        for b in (0, 1):
            wait_idx(b)
            process(b)
            issue_store(b, b)
            issue_idx(b + 2, b)

        def pair_body(g2, carry):
            for b in (0, 1):
                g = g2 * 2 + b
                wait_idx(b)
                wait_store(b)
                process(b)
                issue_store(g, b)
                nxt = jnp.minimum(g + 2, ROWS_PER_WORKER - 1)
                issue_idx(nxt, b)
            return carry

        lax.fori_loop(1, ROWS_PER_WORKER // 2, pair_body, 0)

        for b in (0, 1):
            wait_idx(b)
            wait_store(b)

    return sc_embed


_SC_EMBED = _make_sc_kernel()


@jax.jit
def kernel(visit_orders, pe):
    vo_flat = visit_orders.astype(jnp.int32).reshape(BATCH * HIST)
    out = _SC_EMBED(vo_flat, pe)
    return out.reshape(BATCH, HIST, EMB)


# speedup vs baseline: 1.1321x; 1.1321x over previous
import functools

import jax
import jax.numpy as jnp
from jax import lax
from jax.experimental import pallas as pl
from jax.experimental.pallas import tpu as pltpu
from jax.experimental.pallas import tpu_sc as plsc

EMB = 128
MAX_LEN = 512
BATCH = 16384
HIST = 200

NUM_CORES = 2
NUM_SUBCORES = 16
NUM_WORKERS = NUM_CORES * NUM_SUBCORES
LANES = 16

ROWS_PER_WORKER = BATCH // NUM_WORKERS
SLICE_OFFS = tuple(range(0, HIST - LANES + 1, LANES)) + (HIST - LANES,)


def _vgather(v, idx):
    dnums = lax.GatherDimensionNumbers(
        offset_dims=(), collapsed_slice_dims=(0,), start_index_map=(0,))
    return lax.gather(v, idx[:, None], dnums, (1,),
                      mode=lax.GatherScatterMode.PROMISE_IN_BOUNDS)


def _make_sc_kernel():
    mesh = plsc.VectorSubcoreMesh(core_axis_name="c", subcore_axis_name="s")

    @functools.partial(
        pl.kernel,
        mesh=mesh,
        compiler_params=pltpu.CompilerParams(needs_layout_passes=False),
        out_type=jax.ShapeDtypeStruct((BATCH * HIST, EMB), jnp.float32),
        scratch_types=[
            pltpu.VMEM((MAX_LEN, EMB), jnp.float32),
            pltpu.VMEM((HIST,), jnp.int32),
            pltpu.VMEM((HIST,), jnp.int32),
            pltpu.VMEM((HIST, EMB), jnp.float32),
            pltpu.VMEM((HIST, EMB), jnp.float32),
            pltpu.VMEM((HIST,), jnp.int32),
            pltpu.SemaphoreType.DMA,
            pltpu.SemaphoreType.DMA,
            pltpu.SemaphoreType.DMA,
            pltpu.SemaphoreType.DMA,
        ],
    )
    def sc_embed(vo_hbm, pe_hbm, out_hbm, pe_l, idx0, idx1, out0, out1,
                 norm, si0, si1, so0, so1):
        wid = lax.axis_index("s") * NUM_CORES + lax.axis_index("c")
        wbase = wid * ROWS_PER_WORKER
        zeros16 = jnp.zeros((LANES,), jnp.int32)
        iota16 = lax.iota(jnp.int32, LANES)
        cols = [iota16 + c * LANES for c in range(EMB // LANES)]
        idx_b = (idx0, idx1)
        out_b = (out0, out1)
        sem_i = (si0, si1)
        sem_o = (so0, so1)

        pltpu.sync_copy(pe_hbm, pe_l)

        def issue_idx(chunk, b):
            pltpu.async_copy(
                vo_hbm.at[pl.ds((wbase + chunk) * HIST, HIST)], idx_b[b],
                sem_i[b])

        def wait_idx(b):
            pltpu.make_async_copy(
                vo_hbm.at[pl.ds(0, HIST)], idx_b[b], sem_i[b]).wait()

        def issue_store(chunk, b):
            pltpu.async_copy(
                out_b[b], out_hbm.at[pl.ds((wbase + chunk) * HIST, HIST)],
                sem_o[b])

        def wait_store(b):
            pltpu.make_async_copy(
                out_b[b], out_hbm.at[pl.ds(0, HIST)], sem_o[b]).wait()

        def process(b):
            raw = idx_b[b]
            out_ref = out_b[b]
            f0 = _vgather(raw[pl.ds(0, LANES)], zeros16)
            for lo in SLICE_OFFS:
                v = raw[pl.ds(lo, LANES)]
                norm[pl.ds(lo, LANES)] = jnp.clip(v - f0, 0, MAX_LEN - 1)

            @plsc.parallel_loop(0, HIST, unroll=4)
            def entry_body(e):
                e_vec = jnp.full((LANES,), e, jnp.int32)
                r = plsc.load_gather(norm, [e_vec])
                for c in range(EMB // LANES):
                    val = plsc.load_gather(pe_l, [r, cols[c]])
                    plsc.store_scatter(out_ref, [e_vec, cols[c]], val)

        issue_idx(0, 0)
        issue_idx(1, 1)
        for b in (0, 1):
            wait_idx(b)
            process(b)
            issue_store(b, b)
            issue_idx(b + 2, b)

        def pair_body(g2, carry):
            for b in (0, 1):
                g = g2 * 2 + b
                wait_idx(b)
                wait_store(b)
                process(b)
                issue_store(g, b)
                nxt = jnp.minimum(g + 2, ROWS_PER_WORKER - 1)
                issue_idx(nxt, b)
            return carry

        lax.fori_loop(1, ROWS_PER_WORKER // 2, pair_body, 0)

        for b in (0, 1):
            wait_idx(b)
            wait_store(b)

    return sc_embed


_SC_EMBED = _make_sc_kernel()


@jax.jit
def kernel(visit_orders, pe):
    vo_flat = visit_orders.astype(jnp.int32).reshape(BATCH * HIST)
    out = _SC_EMBED(vo_flat, pe)
    return out.reshape(BATCH, HIST, EMB)


# speedup vs baseline: 1.1376x; 1.0049x over previous
import functools

import jax
import jax.numpy as jnp
from jax import lax
from jax.experimental import pallas as pl
from jax.experimental.pallas import tpu as pltpu
from jax.experimental.pallas import tpu_sc as plsc

EMB = 128
MAX_LEN = 512
BATCH = 16384
HIST = 200

NUM_CORES = 2
NUM_SUBCORES = 16
NUM_WORKERS = NUM_CORES * NUM_SUBCORES
LANES = 16

ROWS_PER_WORKER = BATCH // NUM_WORKERS
SLICE_OFFS = tuple(range(0, HIST - LANES + 1, LANES)) + (HIST - LANES,)


def _vgather(v, idx):
    dnums = lax.GatherDimensionNumbers(
        offset_dims=(), collapsed_slice_dims=(0,), start_index_map=(0,))
    return lax.gather(v, idx[:, None], dnums, (1,),
                      mode=lax.GatherScatterMode.PROMISE_IN_BOUNDS)


def _make_sc_kernel():
    mesh = plsc.VectorSubcoreMesh(core_axis_name="c", subcore_axis_name="s")

    @functools.partial(
        pl.kernel,
        mesh=mesh,
        compiler_params=pltpu.CompilerParams(needs_layout_passes=False),
        out_type=jax.ShapeDtypeStruct((BATCH * HIST, EMB), jnp.float32),
        scratch_types=[
            pltpu.VMEM((MAX_LEN, EMB), jnp.float32),
            pltpu.VMEM((HIST,), jnp.int32),
            pltpu.VMEM((HIST,), jnp.int32),
            pltpu.VMEM((HIST, EMB), jnp.float32),
            pltpu.VMEM((HIST, EMB), jnp.float32),
            pltpu.VMEM((HIST,), jnp.int32),
            pltpu.SemaphoreType.DMA,
            pltpu.SemaphoreType.DMA,
            pltpu.SemaphoreType.DMA,
            pltpu.SemaphoreType.DMA,
        ],
    )
    def sc_embed(vo_hbm, pe_hbm, out_hbm, pe_l, idx0, idx1, out0, out1,
                 norm, si0, si1, so0, so1):
        wid = lax.axis_index("s") * NUM_CORES + lax.axis_index("c")
        wbase = wid * ROWS_PER_WORKER
        zeros16 = jnp.zeros((LANES,), jnp.int32)
        iota16 = lax.iota(jnp.int32, LANES)
        cols = [iota16 + c * LANES for c in range(EMB // LANES)]
        idx_b = (idx0, idx1)
        out_b = (out0, out1)
        sem_i = (si0, si1)
        sem_o = (so0, so1)

        pltpu.sync_copy(pe_hbm, pe_l)

        def issue_idx(chunk, b):
            pltpu.async_copy(
                vo_hbm.at[pl.ds((wbase + chunk) * HIST, HIST)], idx_b[b],
                sem_i[b])

        def wait_idx(b):
            pltpu.make_async_copy(
                vo_hbm.at[pl.ds(0, HIST)], idx_b[b], sem_i[b]).wait()

        def issue_store(chunk, b):
            pltpu.async_copy(
                out_b[b], out_hbm.at[pl.ds((wbase + chunk) * HIST, HIST)],
                sem_o[b])

        def wait_store(b):
            pltpu.make_async_copy(
                out_b[b], out_hbm.at[pl.ds(0, HIST)], sem_o[b]).wait()

        def process(b):
            raw = idx_b[b]
            out_ref = out_b[b]
            f0 = _vgather(raw[pl.ds(0, LANES)], zeros16)
            for lo in SLICE_OFFS:
                v = raw[pl.ds(lo, LANES)]
                norm[pl.ds(lo, LANES)] = jnp.clip(v - f0, 0, MAX_LEN - 1)

            @plsc.parallel_loop(0, HIST, unroll=5)
            def entry_body(e):
                e_vec = jnp.full((LANES,), e, jnp.int32)
                r = plsc.load_gather(norm, [e_vec])
                for c in range(EMB // LANES):
                    val = plsc.load_gather(pe_l, [r, cols[c]])
                    plsc.store_scatter(out_ref, [e_vec, cols[c]], val)

        issue_idx(0, 0)
        issue_idx(1, 1)
        for b in (0, 1):
            wait_idx(b)
            process(b)
            issue_store(b, b)
            issue_idx(b + 2, b)

        def pair_body(g2, carry):
            for b in (0, 1):
                g = g2 * 2 + b
                wait_idx(b)
                wait_store(b)
                process(b)
                issue_store(g, b)
                nxt = jnp.minimum(g + 2, ROWS_PER_WORKER - 1)
                issue_idx(nxt, b)
            return carry

        lax.fori_loop(1, ROWS_PER_WORKER // 2, pair_body, 0)

        for b in (0, 1):
            wait_idx(b)
            wait_store(b)

    return sc_embed


_SC_EMBED = _make_sc_kernel()


@jax.jit
def kernel(visit_orders, pe):
    vo_flat = visit_orders.astype(jnp.int32).reshape(BATCH * HIST)
    out = _SC_EMBED(vo_flat, pe)
    return out.reshape(BATCH, HIST, EMB)
